# Initial kernel scaffold; baseline (speedup 1.0000x reference)
import jax
import jax.numpy as jnp
from jax.experimental import pallas as pl


def kernel(positive_triplets, negative_triplets, Eh, Et, Rt, Rc, Rcl, Rp):
    raise NotImplementedError("write your pallas kernel here")


# speedup vs baseline: 2.2168x; 2.2168x over previous
import functools

import jax
import jax.numpy as jnp
from jax import lax
from jax.experimental import pallas as pl
from jax.experimental.pallas import tpu as pltpu
from jax.experimental.pallas import tpu_sc as plsc

_NC = 2
_NS = 16
_NW = _NC * _NS
_L = 16
_DIM = 64
_NV = _DIM // _L
_C = 128


def _rsqrt(x):
    i = plsc.bitcast(x, jnp.int32)
    i = jnp.int32(0x5F3759DF) - lax.shift_right_logical(i, 1)
    y = plsc.bitcast(i, jnp.float32)
    for _ in range(3):
        y = y * (jnp.float32(1.5) - jnp.float32(0.5) * x * y * y)
    return y


def _sqrt(x):
    return x * _rsqrt(jnp.maximum(x, jnp.float32(1e-30)))


def _make_kernel(B):
    rows_per_tile = B // _NW
    chunks = rows_per_tile // _C
    groups = _C // _L
    nvec = rows_per_tile // _L

    def body(Eh, Et, Rt, Rc, Rcl, Rp, idx, loss_o, pd_o, nd_o,
             idxv, hbuf, tbuf, rbuf, part, dbuf, lbuf, sem):
        wid = lax.axis_index("s") * _NC + lax.axis_index("c")
        base = wid * rows_per_tile

        for tab in range(12):
            pltpu.sync_copy(idx.at[tab, pl.ds(wid * chunks, chunks)],
                            idxv.at[tab])

        lane = lax.iota(jnp.int32, _L)

        def chunk_body(half, rel_tabs, h_tab, t_tab):
            def one_chunk(c, _):
                cp_h = pltpu.async_copy(Eh.at[idxv.at[h_tab, c]], hbuf, sem)
                cp_t = pltpu.async_copy(Et.at[idxv.at[t_tab, c]], tbuf, sem)
                cp_r0 = pltpu.async_copy(Rt.at[idxv.at[rel_tabs[0], c]],
                                         rbuf.at[0], sem)
                cp_r1 = pltpu.async_copy(Rc.at[idxv.at[rel_tabs[1], c]],
                                         rbuf.at[1], sem)
                cp_r2 = pltpu.async_copy(Rcl.at[idxv.at[rel_tabs[2], c]],
                                         rbuf.at[2], sem)
                cp_r3 = pltpu.async_copy(Rp.at[idxv.at[rel_tabs[3], c]],
                                         rbuf.at[3], sem)
                cp_h.wait()
                cp_t.wait()
                cp_r0.wait()
                cp_r1.wait()
                cp_r2.wait()
                cp_r3.wait()

                def row_body(i, _):
                    ssh = sst = rr = hr = tr = ht = None
                    for k in range(_NV):
                        sl = pl.ds(k * _L, _L)
                        h = hbuf[i, sl]
                        t = tbuf[i, sl]
                        r = (rbuf[0, i, sl] + rbuf[1, i, sl]
                             + rbuf[2, i, sl] + rbuf[3, i, sl])
                        if k == 0:
                            ssh, sst, rr = h * h, t * t, r * r
                            hr, tr, ht = h * r, t * r, h * t
                        else:
                            ssh += h * h
                            sst += t * t
                            rr += r * r
                            hr += h * r
                            tr += t * r
                            ht += h * t
                    part[pl.ds(i * 6 * _L, _L)] = ssh
                    part[pl.ds((i * 6 + 1) * _L, _L)] = sst
                    part[pl.ds((i * 6 + 2) * _L, _L)] = rr
                    part[pl.ds((i * 6 + 3) * _L, _L)] = hr
                    part[pl.ds((i * 6 + 4) * _L, _L)] = tr
                    part[pl.ds((i * 6 + 5) * _L, _L)] = ht
                    return _

                lax.fori_loop(0, _C, row_body, None)

                def group_body(g, _):
                    rowv = (g * _L + lane) * (6 * _L)
                    sums = []
                    for d in range(6):
                        acc = plsc.load_gather(part, [rowv + d * _L])
                        for l in range(1, _L):
                            acc += plsc.load_gather(part, [rowv + (d * _L + l)])
                        sums.append(acc)
                    ssh, sst, rr, hr, tr, ht = sums
                    a = _rsqrt(jnp.maximum(ssh, jnp.float32(1e-30)))
                    b = _rsqrt(jnp.maximum(sst, jnp.float32(1e-30)))
                    two = jnp.float32(2.0)
                    ssd = two + rr + two * a * hr - two * b * tr \
                        - two * (a * b) * ht
                    ssd = jnp.maximum(ssd, jnp.float32(0.0))
                    dbuf[half, pl.ds(c * _C + g * _L, _L)] = _sqrt(ssd)
                    return _

                lax.fori_loop(0, groups, group_body, None)
                return _

            lax.fori_loop(0, chunks, one_chunk, None)

        chunk_body(0, (1, 2, 3, 4), 0, 5)
        chunk_body(1, (7, 8, 9, 10), 6, 11)

        def loss_body(j, _):
            sl = pl.ds(j * _L, _L)
            p = dbuf[0, sl]
            n = dbuf[1, sl]
            lbuf[sl] = jnp.maximum(p - n + jnp.float32(1.0), jnp.float32(0.0))
            return _

        lax.fori_loop(0, nvec, loss_body, None)

        out_sl = pl.ds(base, rows_per_tile)
        pltpu.sync_copy(lbuf, loss_o.at[out_sl])
        pltpu.sync_copy(dbuf.at[0], pd_o.at[out_sl])
        pltpu.sync_copy(dbuf.at[1], nd_o.at[out_sl])

    f32 = jnp.float32
    return pl.kernel(
        body,
        out_type=(
            jax.ShapeDtypeStruct((B,), f32),
            jax.ShapeDtypeStruct((B,), f32),
            jax.ShapeDtypeStruct((B,), f32),
        ),
        mesh=plsc.VectorSubcoreMesh(core_axis_name="c", subcore_axis_name="s"),
        compiler_params=pltpu.CompilerParams(
            needs_layout_passes=False, use_tc_tiling_on_sc=False),
        scratch_types=[
            pltpu.VMEM((12, chunks, _C), jnp.int32),
            pltpu.VMEM((_C, _DIM), f32),
            pltpu.VMEM((_C, _DIM), f32),
            pltpu.VMEM((4, _C, _DIM), f32),
            pltpu.VMEM((6 * _C * _L,), f32),
            pltpu.VMEM((2, rows_per_tile), f32),
            pltpu.VMEM((rows_per_tile,), f32),
            pltpu.SemaphoreType.DMA,
        ],
    )


@jax.jit
def kernel(positive_triplets, negative_triplets, Eh, Et, Rt, Rc, Rcl, Rp):
    B = positive_triplets.shape[0]
    cols = jnp.concatenate(
        [positive_triplets.T, negative_triplets.T], axis=0).astype(jnp.int32)
    idx = cols.reshape(12, B // _C, _C)
    loss, pd, nd = _make_kernel(B)(Eh, Et, Rt, Rc, Rcl, Rp, idx)
    return loss, pd, nd


# speedup vs baseline: 5.3522x; 2.4144x over previous
import functools

import jax
import jax.numpy as jnp
from jax import lax
from jax.experimental import pallas as pl
from jax.experimental.pallas import tpu as pltpu
from jax.experimental.pallas import tpu_sc as plsc

_NC = 2
_NS = 16
_NW = _NC * _NS
_L = 16
_DIM = 64
_NV = _DIM // _L
_C = 128
_TBLK = 20480


def _rsqrt(x):
    i = plsc.bitcast(x, jnp.int32)
    i = jnp.int32(0x5F3759DF) - lax.shift_right_logical(i, 1)
    y = plsc.bitcast(i, jnp.float32)
    for _ in range(3):
        y = y * (jnp.float32(1.5) - jnp.float32(0.5) * x * y * y)
    return y


def _sqrt(x):
    return x * _rsqrt(jnp.maximum(x, jnp.float32(1e-30)))


def _tn_body(h_ref, t_ref, o_ref):
    h = h_ref[...]
    hn = h * lax.rsqrt(jnp.sum(h * h, axis=0, keepdims=True))
    t = t_ref[...]
    tn = t * lax.rsqrt(jnp.sum(t * t, axis=0, keepdims=True))
    o_ref[:, 0:_DIM] = jnp.swapaxes(hn, 0, 1)
    o_ref[:, _DIM:2 * _DIM] = jnp.swapaxes(tn, 0, 1)


def _transpose_normalize(ehT, etT, n_rows):
    grid = (n_rows + _TBLK - 1) // _TBLK
    return pl.pallas_call(
        _tn_body,
        grid=(grid,),
        in_specs=[pl.BlockSpec((_DIM, _TBLK), lambda i: (0, i)),
                  pl.BlockSpec((_DIM, _TBLK), lambda i: (0, i))],
        out_specs=pl.BlockSpec((_TBLK, 2 * _DIM), lambda i: (i, 0)),
        out_shape=jax.ShapeDtypeStruct((n_rows, 2 * _DIM), jnp.float32),
    )(ehT, etT)


def _make_sc_kernel(B):
    rows_per_tile = B // _NW
    chunks = rows_per_tile // _C
    groups = _C // _L
    nvec = rows_per_tile // _L

    def body(ET, Rt, Rc, Rcl, Rp, idx, loss_o, pd_o, nd_o,
             idxv, hbuf, tbuf, rbuf, part, dbuf, lbuf, sem):
        wid = lax.axis_index("s") * _NC + lax.axis_index("c")
        base = wid * rows_per_tile

        for tab in range(12):
            pltpu.sync_copy(idx.at[tab, pl.ds(wid * chunks, chunks)],
                            idxv.at[tab])

        lane = lax.iota(jnp.int32, _L)

        def chunk_body(half, rel_tabs, h_tab, t_tab):
            def one_chunk(c, _):
                cp_h = pltpu.async_copy(ET.at[idxv.at[h_tab, c]], hbuf, sem)
                cp_t = pltpu.async_copy(ET.at[idxv.at[t_tab, c]], tbuf, sem)
                cp_r0 = pltpu.async_copy(Rt.at[idxv.at[rel_tabs[0], c]],
                                         rbuf.at[0], sem)
                cp_r1 = pltpu.async_copy(Rc.at[idxv.at[rel_tabs[1], c]],
                                         rbuf.at[1], sem)
                cp_r2 = pltpu.async_copy(Rcl.at[idxv.at[rel_tabs[2], c]],
                                         rbuf.at[2], sem)
                cp_r3 = pltpu.async_copy(Rp.at[idxv.at[rel_tabs[3], c]],
                                         rbuf.at[3], sem)
                cp_h.wait()
                cp_t.wait()
                cp_r0.wait()
                cp_r1.wait()
                cp_r2.wait()
                cp_r3.wait()

                def row_body(i, _):
                    ssd = None
                    for k in range(_NV):
                        sl = pl.ds(k * _L, _L)
                        tsl = pl.ds(_DIM + k * _L, _L)
                        d = (hbuf[i, sl] - tbuf[i, tsl]
                             + rbuf[0, i, sl] + rbuf[1, i, sl]
                             + rbuf[2, i, sl] + rbuf[3, i, sl])
                        ssd = d * d if k == 0 else ssd + d * d
                    part[pl.ds(i * _L, _L)] = ssd
                    return _

                lax.fori_loop(0, _C, row_body, None)

                def group_body(g, _):
                    rowv = (g * _L + lane) * _L
                    acc = plsc.load_gather(part, [rowv])
                    for l in range(1, _L):
                        acc += plsc.load_gather(part, [rowv + l])
                    dbuf[half, pl.ds(c * _C + g * _L, _L)] = _sqrt(acc)
                    return _

                lax.fori_loop(0, groups, group_body, None)
                return _

            lax.fori_loop(0, chunks, one_chunk, None)

        chunk_body(0, (1, 2, 3, 4), 0, 5)
        chunk_body(1, (7, 8, 9, 10), 6, 11)

        def loss_body(j, _):
            sl = pl.ds(j * _L, _L)
            p = dbuf[0, sl]
            n = dbuf[1, sl]
            lbuf[sl] = jnp.maximum(p - n + jnp.float32(1.0), jnp.float32(0.0))
            return _

        lax.fori_loop(0, nvec, loss_body, None)

        out_sl = pl.ds(base, rows_per_tile)
        pltpu.sync_copy(lbuf, loss_o.at[out_sl])
        pltpu.sync_copy(dbuf.at[0], pd_o.at[out_sl])
        pltpu.sync_copy(dbuf.at[1], nd_o.at[out_sl])

    f32 = jnp.float32
    return pl.kernel(
        body,
        out_type=(
            jax.ShapeDtypeStruct((B,), f32),
            jax.ShapeDtypeStruct((B,), f32),
            jax.ShapeDtypeStruct((B,), f32),
        ),
        mesh=plsc.VectorSubcoreMesh(core_axis_name="c", subcore_axis_name="s"),
        compiler_params=pltpu.CompilerParams(
            needs_layout_passes=False, use_tc_tiling_on_sc=False),
        scratch_types=[
            pltpu.VMEM((12, chunks, _C), jnp.int32),
            pltpu.VMEM((_C, 2 * _DIM), f32),
            pltpu.VMEM((_C, 2 * _DIM), f32),
            pltpu.VMEM((4, _C, _DIM), f32),
            pltpu.VMEM((_C * _L,), f32),
            pltpu.VMEM((2, rows_per_tile), f32),
            pltpu.VMEM((rows_per_tile,), f32),
            pltpu.SemaphoreType.DMA,
        ],
    )


@jax.jit
def kernel(positive_triplets, negative_triplets, Eh, Et, Rt, Rc, Rcl, Rp):
    B = positive_triplets.shape[0]
    n_ent = Eh.shape[0]
    ENt = _transpose_normalize(Eh.T, Et.T, n_ent)
    cols = jnp.concatenate(
        [positive_triplets.T, negative_triplets.T], axis=0).astype(jnp.int32)
    idx = cols.reshape(12, B // _C, _C)
    loss, pd, nd = _make_sc_kernel(B)(ENt, Rt, Rc, Rcl, Rp, idx)
    return loss, pd, nd


# speedup vs baseline: 5.3543x; 1.0004x over previous
import jax
import jax.numpy as jnp
from jax import lax
from jax.experimental import pallas as pl
from jax.experimental.pallas import tpu as pltpu
from jax.experimental.pallas import tpu_sc as plsc

_NC = 2
_NS = 16
_NW = _NC * _NS
_L = 16
_DIM = 64
_NV = _DIM // _L
_C = 128
_TBLK = 20480


def _rsqrt(x):
    i = plsc.bitcast(x, jnp.int32)
    i = jnp.int32(0x5F3759DF) - lax.shift_right_logical(i, 1)
    y = plsc.bitcast(i, jnp.float32)
    for _ in range(3):
        y = y * (jnp.float32(1.5) - jnp.float32(0.5) * x * y * y)
    return y


def _sqrt(x):
    return x * _rsqrt(jnp.maximum(x, jnp.float32(1e-30)))


def _tn_body(h_ref, t_ref, o_ref):
    h = h_ref[...]
    hn = h * lax.rsqrt(jnp.sum(h * h, axis=0, keepdims=True))
    t = t_ref[...]
    tn = t * lax.rsqrt(jnp.sum(t * t, axis=0, keepdims=True))
    o_ref[:, 0:_DIM] = jnp.swapaxes(hn, 0, 1)
    o_ref[:, _DIM:2 * _DIM] = jnp.swapaxes(tn, 0, 1)


def _transpose_normalize(ehT, etT, n_rows):
    grid = (n_rows + _TBLK - 1) // _TBLK
    return pl.pallas_call(
        _tn_body,
        grid=(grid,),
        in_specs=[pl.BlockSpec((_DIM, _TBLK), lambda i: (0, i)),
                  pl.BlockSpec((_DIM, _TBLK), lambda i: (0, i))],
        out_specs=pl.BlockSpec((_TBLK, 2 * _DIM), lambda i: (i, 0)),
        out_shape=jax.ShapeDtypeStruct((n_rows, 2 * _DIM), jnp.float32),
    )(ehT, etT)


def _make_sc_kernel(B):
    rows_per_tile = B // _NW
    chunks = rows_per_tile // _C
    groups = _C // _L
    nvec = rows_per_tile // _L

    def body(ET, Rt, Rc, Rcl, Rp, idx, loss_o, pd_o, nd_o,
             idxv, hbuf, tbuf, rbuf, part, dbuf, lbuf, sem):
        wid = lax.axis_index("s") * _NC + lax.axis_index("c")
        base = wid * rows_per_tile

        for tab in range(12):
            pltpu.sync_copy(idx.at[tab, pl.ds(wid * chunks, chunks)],
                            idxv.at[tab])

        lane = lax.iota(jnp.int32, _L)

        def chunk_body(half, rel_tabs, h_tab, t_tab):
            def one_chunk(c, _):
                cp_h = pltpu.async_copy(ET.at[idxv.at[h_tab, c]], hbuf, sem)
                cp_t = pltpu.async_copy(ET.at[idxv.at[t_tab, c]], tbuf, sem)
                cp_r0 = pltpu.async_copy(Rt.at[idxv.at[rel_tabs[0], c]],
                                         rbuf.at[0], sem)
                cp_r1 = pltpu.async_copy(Rc.at[idxv.at[rel_tabs[1], c]],
                                         rbuf.at[1], sem)
                cp_r2 = pltpu.async_copy(Rcl.at[idxv.at[rel_tabs[2], c]],
                                         rbuf.at[2], sem)
                cp_r3 = pltpu.async_copy(Rp.at[idxv.at[rel_tabs[3], c]],
                                         rbuf.at[3], sem)
                cp_h.wait()
                cp_t.wait()
                cp_r0.wait()
                cp_r1.wait()
                cp_r2.wait()
                cp_r3.wait()

                def row_body(i, _):
                    ssd = None
                    for k in range(_NV):
                        sl = pl.ds(k * _L, _L)
                        tsl = pl.ds(_DIM + k * _L, _L)
                        d = (hbuf[i, sl] - tbuf[i, tsl]
                             + rbuf[0, i, sl] + rbuf[1, i, sl]
                             + rbuf[2, i, sl] + rbuf[3, i, sl])
                        ssd = d * d if k == 0 else ssd + d * d
                    part[pl.ds(i * _L, _L)] = ssd
                    return _

                lax.fori_loop(0, _C, row_body, None)

                def group_body(g, _):
                    rowv = (g * _L + lane) * _L
                    acc = plsc.load_gather(part, [rowv])
                    for l in range(1, _L):
                        acc += plsc.load_gather(part, [rowv + l])
                    dbuf[half, pl.ds(c * _C + g * _L, _L)] = _sqrt(acc)
                    return _

                lax.fori_loop(0, groups, group_body, None)
                return _

            lax.fori_loop(0, chunks, one_chunk, None)

        chunk_body(0, (1, 2, 3, 4), 0, 5)
        chunk_body(1, (7, 8, 9, 10), 6, 11)

        def loss_body(j, _):
            sl = pl.ds(j * _L, _L)
            p = dbuf[0, sl]
            n = dbuf[1, sl]
            lbuf[sl] = jnp.maximum(p - n + jnp.float32(1.0), jnp.float32(0.0))
            return _

        lax.fori_loop(0, nvec, loss_body, None)

        out_sl = pl.ds(base, rows_per_tile)
        pltpu.sync_copy(lbuf, loss_o.at[out_sl])
        pltpu.sync_copy(dbuf.at[0], pd_o.at[out_sl])
        pltpu.sync_copy(dbuf.at[1], nd_o.at[out_sl])

    f32 = jnp.float32
    return pl.kernel(
        body,
        out_type=(
            jax.ShapeDtypeStruct((B,), f32),
            jax.ShapeDtypeStruct((B,), f32),
            jax.ShapeDtypeStruct((B,), f32),
        ),
        mesh=plsc.VectorSubcoreMesh(core_axis_name="c", subcore_axis_name="s"),
        compiler_params=pltpu.CompilerParams(
            needs_layout_passes=False, use_tc_tiling_on_sc=False),
        scratch_types=[
            pltpu.VMEM((12, chunks, _C), jnp.int32),
            pltpu.VMEM((_C, 2 * _DIM), f32),
            pltpu.VMEM((_C, 2 * _DIM), f32),
            pltpu.VMEM((4, _C, _DIM), f32),
            pltpu.VMEM((_C * _L,), f32),
            pltpu.VMEM((2, rows_per_tile), f32),
            pltpu.VMEM((rows_per_tile,), f32),
            pltpu.SemaphoreType.DMA,
        ],
    )


@jax.jit
def kernel(positive_triplets, negative_triplets, Eh, Et, Rt, Rc, Rcl, Rp):
    B = positive_triplets.shape[0]
    n_ent = Eh.shape[0]
    ENt = _transpose_normalize(Eh.T, Et.T, n_ent)
    cols = jnp.concatenate(
        [positive_triplets.T, negative_triplets.T], axis=0).astype(jnp.int32)
    idx = cols.reshape(12, B // _C, _C)
    loss, pd, nd = _make_sc_kernel(B)(ENt, Rt, Rc, Rcl, Rp, idx)
    return loss, pd, nd
